# Initial kernel scaffold; baseline (speedup 1.0000x reference)
import jax
import jax.numpy as jnp
from jax.experimental import pallas as pl


def kernel(class_logits, box_regression, anchors, res_scores, post_max_into_pre_max):
    raise NotImplementedError("write your pallas kernel here")


# speedup vs baseline: 84.0798x; 84.0798x over previous
import functools

import jax
import jax.numpy as jnp
from jax import lax
from jax.experimental import pallas as pl
from jax.experimental.pallas import tpu as pltpu
from jax.experimental.pallas import tpu_sc as plsc

N = 20000
LANES = 128
ROWS = 160
NPAD = ROWS * LANES
K = 2000
KPAD = 2048
BLK = 256
NCH = 16
POST = 100
NEG = -3.0e38

f32 = jnp.float32
i32 = jnp.int32
u32 = jnp.uint32

C_XG, C_YG, C_ZG, C_WG, C_LG, C_HG, C_RG, C_RW = 0, 1, 2, 3, 4, 5, 6, 7
C_X2, C_Y2, C_RA, C_ST, C_SR, C_IDX = 8, 9, 10, 11, 12, 13


def _sortkey(s, valid):
    u = lax.bitcast_convert_type(s, u32)
    key = jnp.where(u >= u32(0x80000000), ~u, u | u32(0x80000000))
    return jnp.where(valid, key, u32(0))


def _kth_key(key, kwant):
    t = u32(0)
    for b in range(31, -1, -1):
        cand = t | u32(1 << b)
        cnt = jnp.sum((key >= cand).astype(i32))
        t = jnp.where(cnt >= kwant, cand, t)
    return t


def _excl_cumsum(x, l_incl, l_strict_rows):
    incl = jnp.dot(x, l_incl, preferred_element_type=f32)
    rowtot = jnp.sum(x, axis=1, keepdims=True)
    prev_rows = jnp.dot(l_strict_rows, rowtot, preferred_element_type=f32)
    return incl - x + prev_rows


def _select_indices(key, flatidx_f, l_incl, l_strict_rows, slot_iota):
    t = _kth_key(key, K)
    gt = key > t
    eq = key == t
    c_gt = jnp.sum(gt.astype(f32))
    need = f32(K) - c_gt
    tie_rank = _excl_cumsum(eq.astype(f32), l_incl, l_strict_rows)
    sel = gt | (eq & (tie_rank < need))
    sel_f = sel.astype(f32)
    pos = _excl_cumsum(sel_f, l_incl, l_strict_rows).astype(i32)
    acc = jnp.zeros((KPAD, 1), f32)
    for b in range(ROWS // 8):
        sl = slice(b * 8, (b + 1) * 8)
        posb = jnp.reshape(pos[sl, :], (1, 8 * LANES))
        selb = jnp.reshape(sel_f[sl, :], (1, 8 * LANES))
        vb = jnp.reshape(flatidx_f[sl, :], (1, 8 * LANES))
        a = (slot_iota == posb).astype(f32) * selb
        acc = acc + lax.dot_general(a, vb, (((1,), (1,)), ((), ())),
                                    preferred_element_type=f32)
    return acc


def _prep_body(resid_ref, logit_ref, res_ref, reg_ref, anc_ref,
               table_ref, srct_ref, srcr_ref):
    resid = resid_ref[0, 0]
    logit = logit_ref[...]
    res = res_ref[...]
    xa, ya, za = anc_ref[0], anc_ref[1], anc_ref[2]
    wa, la, ha, ra = anc_ref[3], anc_ref[4], anc_ref[5], anc_ref[6]
    r0, r1, r2 = reg_ref[0], reg_ref[1], reg_ref[2]
    r3, r4, r5, r6 = reg_ref[3], reg_ref[4], reg_ref[5], reg_ref[6]

    diag = jnp.sqrt(la * la + wa * wa)
    xg = r0 / 10.0 * diag + xa
    yg = r1 / 10.0 * diag + ya
    zg = r2 / 10.0 * ha + za
    wg = jnp.exp(r3 / 5.0) * wa
    lg = jnp.exp(r4 / 5.0) * la
    hg = jnp.exp(r5 / 5.0) * ha
    rg = r6 / 10.0 + ra
    rw = jnp.arctan2(jnp.sin(rg), jnp.cos(rg))
    x2 = r0 / 10.0 * wa + xa
    y2 = r1 / 10.0 * la + ya

    score_t = jax.nn.sigmoid(jax.nn.sigmoid(logit)) + resid
    score_r = res + resid

    row_i = lax.broadcasted_iota(i32, (ROWS, LANES), 0)
    lane_i = lax.broadcasted_iota(i32, (ROWS, LANES), 1)
    flat = row_i * LANES + lane_i
    valid = flat < N
    flat_f = flat.astype(f32)

    for c, v in ((C_XG, xg), (C_YG, yg), (C_ZG, zg), (C_WG, wg), (C_LG, lg),
                 (C_HG, hg), (C_RG, rg), (C_RW, rw), (C_X2, x2), (C_Y2, y2),
                 (C_RA, ra), (C_ST, score_t), (C_SR, score_r), (C_IDX, flat_f),
                 (14, jnp.zeros((ROWS, LANES), f32)),
                 (15, jnp.zeros((ROWS, LANES), f32))):
        table_ref[c] = v

    li_r = lax.broadcasted_iota(i32, (LANES, LANES), 0)
    li_c = lax.broadcasted_iota(i32, (LANES, LANES), 1)
    l_incl = (li_r <= li_c).astype(f32)
    rr = lax.broadcasted_iota(i32, (ROWS, ROWS), 0)
    rc = lax.broadcasted_iota(i32, (ROWS, ROWS), 1)
    l_strict = (rc < rr).astype(f32)
    slot_iota = lax.broadcasted_iota(i32, (KPAD, 1), 0)

    key_t = _sortkey(score_t, valid)
    key_r = _sortkey(score_r, valid)
    srct_ref[...] = _select_indices(key_t, flat_f, l_incl, l_strict,
                                    slot_iota).astype(i32)
    srcr_ref[...] = _select_indices(key_r, flat_f, l_incl, l_strict,
                                    slot_iota).astype(i32)


def _prep(resid, logit, res, reg, anc):
    return pl.pallas_call(
        _prep_body,
        out_shape=[
            jax.ShapeDtypeStruct((NCH, ROWS, LANES), f32),
            jax.ShapeDtypeStruct((KPAD, 1), i32),
            jax.ShapeDtypeStruct((KPAD, 1), i32),
        ],
    )(resid, logit, res, reg, anc)


def _gather_rows(table2d, idx):
    b_total = idx.shape[0]
    nw = 32
    b_per_w = b_total // nw
    mesh = plsc.VectorSubcoreMesh(core_axis_name="c", subcore_axis_name="s")

    @functools.partial(
        pl.kernel, mesh=mesh,
        out_type=jax.ShapeDtypeStruct((b_total, NCH), f32),
        compiler_params=pltpu.CompilerParams(use_tc_tiling_on_sc=False),
        scratch_types=[
            pltpu.VMEM((b_per_w,), i32),
            pltpu.VMEM((b_per_w, NCH), f32),
            pltpu.SemaphoreType.DMA,
        ],
    )
    def k(table_hbm, idx_hbm, out_hbm, idx_v, rows_v, sem):
        wid = lax.axis_index("s") * 2 + lax.axis_index("c")
        base = wid * b_per_w
        pltpu.sync_copy(idx_hbm.at[pl.ds(base, b_per_w)], idx_v)
        pltpu.async_copy(table_hbm.at[idx_v], rows_v, sem).wait()
        pltpu.sync_copy(rows_v, out_hbm.at[pl.ds(base, b_per_w)])

    return k(table2d, idx)


def _row(t, c):
    return t[c:c + 1, :]


def _iou_prec_block(colT, rowpre, sch, blk):
    x1j, x2j, y1j, y2j, areaj, sj, ij = rowpre
    sl = slice(blk * BLK, (blk + 1) * BLK)
    xi = colT[sl, 0:1]
    yi = colT[sl, 1:2]
    wi = colT[sl, 2:3]
    li = colT[sl, 3:4]
    si = colT[sl, 4:5] if sch is None else colT[sl, sch:sch + 1]
    ii = colT[sl, 5:6]
    x1i = xi - wi * 0.5
    x2i = xi + wi * 0.5
    y1i = yi - li * 0.5
    y2i = yi + li * 0.5
    areai = (x2i - x1i) * (y2i - y1i)
    ix1 = jnp.maximum(x1i, x1j)
    iy1 = jnp.maximum(y1i, y1j)
    ix2 = jnp.minimum(x2i, x2j)
    iy2 = jnp.minimum(y2i, y2j)
    inter = jnp.clip(ix2 - ix1, 0.0) * jnp.clip(iy2 - iy1, 0.0)
    union = areai + areaj - inter
    iou = inter / jnp.maximum(union, 1e-8)
    prec = (si > sj) | ((si == sj) & (ii < ij))
    vi = (lax.broadcasted_iota(i32, (BLK, 1), 0) + blk * BLK) < K
    return ((iou > 0.01) & prec & vi).astype(f32)


def _nms_fixpoint(M_ref, keep_ref):
    keep_ref[...] = jnp.ones((1, KPAD), f32)

    def cond(c):
        return c > 0

    def body(c):
        kv = keep_ref[...]
        supp = jnp.dot(kv, M_ref[...], preferred_element_type=f32)
        knew = (supp < 0.5).astype(f32)
        changed = jnp.sum(jnp.abs(knew - kv))
        keep_ref[...] = knew
        return (changed > 0.0).astype(i32)

    lax.while_loop(cond, body, i32(1))
    return keep_ref[...]


def _rank_of(colsub, rowsub, k_row, valid_row):
    s_row, i_row = rowsub
    m_row = jnp.where((k_row > 0.5) & valid_row, s_row, NEG)
    rank = jnp.zeros((1, KPAD), f32)
    for blk in range(KPAD // BLK):
        sl = slice(blk * BLK, (blk + 1) * BLK)
        eye = (lax.broadcasted_iota(i32, (BLK, BLK), 0)
               == lax.broadcasted_iota(i32, (BLK, BLK), 1))
        k_col = jnp.sum(jnp.where(eye, k_row[:, sl], 0.0), axis=1,
                        keepdims=True)
        v_col = (lax.broadcasted_iota(i32, (BLK, 1), 0) + blk * BLK) < K
        s_col = colsub[sl, 0:1]
        i_col = colsub[sl, 1:2]
        m_col = jnp.where((k_col > 0.5) & v_col, s_col, NEG)
        gt = (m_col > m_row) | ((m_col == m_row) & (i_col < i_row))
        rank = rank + jnp.sum(gt.astype(f32), axis=0, keepdims=True)
    return rank, m_row


def _emit(out_ref, p, rowT, chans, rank, mask_row):
    sel = (lax.broadcasted_iota(i32, (POST + 28, KPAD), 0).astype(f32)
           == rank).astype(f32)
    cols = []
    for c in chans:
        d = _row(rowT, c) * mask_row
        cols.append(jnp.sum(sel * d, axis=1, keepdims=True))
    out_ref[p] = jnp.concatenate(cols, axis=1)


def _nms_body(tc_ref, tr_ref, rc_ref, rr_ref, out_ref, M_ref, keep_ref):
    valid_row = lax.broadcasted_iota(i32, (1, KPAD), 1) < K

    def run(colT_full, rowT, xch, ych, sch, outs):
        colsubM = jnp.concatenate(
            [colT_full[:, xch:xch + 1], colT_full[:, ych:ych + 1],
             colT_full[:, C_WG:C_WG + 1], colT_full[:, C_LG:C_LG + 1],
             colT_full[:, sch:sch + 1], colT_full[:, C_IDX:C_IDX + 1]],
            axis=1)
        xj = _row(rowT, xch)
        yj = _row(rowT, ych)
        wj = _row(rowT, C_WG)
        lj = _row(rowT, C_LG)
        sj = _row(rowT, sch)
        ij = _row(rowT, C_IDX)
        x1j = xj - wj * 0.5
        x2j = xj + wj * 0.5
        y1j = yj - lj * 0.5
        y2j = yj + lj * 0.5
        areaj = (x2j - x1j) * (y2j - y1j)
        rowpre = (x1j, x2j, y1j, y2j, areaj, sj, ij)
        for blk in range(KPAD // BLK):
            sl = slice(blk * BLK, (blk + 1) * BLK)
            M_ref[sl, :] = _iou_prec_block(colsubM, rowpre, 4, blk)
        k_row = _nms_fixpoint(M_ref, keep_ref)
        colsubR = jnp.concatenate(
            [colT_full[:, sch:sch + 1], colT_full[:, C_IDX:C_IDX + 1]], axis=1)
        rank, _ = _rank_of(colsubR, (sj, ij), k_row, valid_row)
        zj = _row(rowT, C_ZG)
        in_rng = ((xj >= 0.0) & (xj <= 70.4) & (yj >= -40.0) & (yj <= 40.0)
                  & (zj >= -2.2) & (zj <= 0.8))
        mask_row = (in_rng & (k_row > 0.5) & valid_row).astype(f32)
        for p, ang in outs:
            _emit(out_ref, p, rowT,
                  (xch, ych, C_ZG, C_WG, C_LG, C_HG, ang, sch), rank, mask_row)

    tc = tc_ref[...]
    rc = rc_ref[...]
    run(tc, tr_ref[...], C_XG, C_YG, C_ST, ((0, C_RG), (2, C_RW)))
    run(tc, tr_ref[...], C_X2, C_Y2, C_ST, ((1, C_RA),))
    run(rc, rr_ref[...], C_XG, C_YG, C_SR, ((3, C_RW),))


def _nms(tc, tr, rc, rr):
    return pl.pallas_call(
        _nms_body,
        out_shape=jax.ShapeDtypeStruct((4, POST + 28, 8), f32),
        scratch_shapes=[
            pltpu.VMEM((KPAD, KPAD), f32),
            pltpu.VMEM((1, KPAD), f32),
        ],
    )(tc, tr, rc, rr)


def kernel(class_logits, box_regression, anchors, res_scores,
           post_max_into_pre_max):
    resid = (jnp.asarray(post_max_into_pre_max, f32) - 2000.0).reshape(1, 1)
    pad = NPAD - N
    logit = jnp.pad(class_logits[:, 0], (0, pad)).reshape(ROWS, LANES)
    res = jnp.pad(res_scores, (0, pad)).reshape(ROWS, LANES)
    reg = jnp.pad(box_regression, ((0, pad), (0, 0))).T.reshape(7, ROWS, LANES)
    anc = jnp.pad(anchors, ((0, pad), (0, 0))).T.reshape(7, ROWS, LANES)

    table, srct, srcr = _prep(resid, logit, res, reg, anc)
    table2d = table.reshape(NCH, NPAD).T
    idx = jnp.concatenate([srct.reshape(-1), srcr.reshape(-1)])
    rows = _gather_rows(table2d, idx)

    tcol = rows[:KPAD]
    rcol = rows[KPAD:]
    out = _nms(tcol, tcol.T, rcol, rcol.T)
    return out[:, :POST, :]


# speedup vs baseline: 87.9173x; 1.0456x over previous
import functools

import jax
import jax.numpy as jnp
from jax import lax
from jax.experimental import pallas as pl
from jax.experimental.pallas import tpu as pltpu
from jax.experimental.pallas import tpu_sc as plsc

N = 20000
LANES = 128
ROWS = 160
NPAD = ROWS * LANES
K = 2000
KPAD = 2048
BLK = 256
NCH = 16
POST = 100
NEG = -3.0e38

f32 = jnp.float32
i32 = jnp.int32
u32 = jnp.uint32

C_XG, C_YG, C_ZG, C_WG, C_LG, C_HG, C_RG, C_RW = 0, 1, 2, 3, 4, 5, 6, 7
C_X2, C_Y2, C_RA, C_ST, C_SR, C_IDX = 8, 9, 10, 11, 12, 13


def _sortkey(s, valid):
    u = lax.bitcast_convert_type(s, u32)
    key = jnp.where(u >= u32(0x80000000), ~u, u | u32(0x80000000))
    return jnp.where(valid, key, u32(0))


def _kth_key2(key_a, key_b, kwant):
    ta = u32(0)
    tb = u32(0)
    for b in range(31, -1, -1):
        ca = ta | u32(1 << b)
        cb = tb | u32(1 << b)
        cnta = jnp.sum((key_a >= ca).astype(i32))
        cntb = jnp.sum((key_b >= cb).astype(i32))
        ta = jnp.where(cnta >= kwant, ca, ta)
        tb = jnp.where(cntb >= kwant, cb, tb)
    return ta, tb


def _col2row(col):
    parts = []
    for blk in range(KPAD // BLK):
        sl = slice(blk * BLK, (blk + 1) * BLK)
        eye = (lax.broadcasted_iota(i32, (BLK, BLK), 0)
               == lax.broadcasted_iota(i32, (BLK, BLK), 1))
        parts.append(jnp.sum(jnp.where(eye, col[sl, :], 0.0), axis=0,
                             keepdims=True))
    return jnp.concatenate(parts, axis=1)


def _excl_cumsum(x, l_incl, l_strict_rows):
    incl = jnp.dot(x, l_incl, preferred_element_type=f32)
    rowtot = jnp.sum(x, axis=1, keepdims=True)
    prev_rows = jnp.dot(l_strict_rows, rowtot, preferred_element_type=f32)
    return incl - x + prev_rows


def _select_indices(key, t, score, flatidx_f, l_incl, l_strict_rows,
                    slot_iota):
    gt = key > t
    eq = key == t
    c_gt = jnp.sum(gt.astype(f32))
    need = f32(K) - c_gt
    tie_rank = _excl_cumsum(eq.astype(f32), l_incl, l_strict_rows)
    sel = gt | (eq & (tie_rank < need))
    sel_f = sel.astype(f32)
    pos = _excl_cumsum(sel_f, l_incl, l_strict_rows).astype(i32)
    src_col = jnp.zeros((KPAD, 1), f32)
    s_col = jnp.zeros((KPAD, 1), f32)
    for b in range(ROWS // 8):
        sl = slice(b * 8, (b + 1) * 8)
        posb = jnp.reshape(pos[sl, :], (1, 8 * LANES))
        selb = jnp.reshape(sel_f[sl, :], (1, 8 * LANES))
        idxb = jnp.reshape(flatidx_f[sl, :], (1, 8 * LANES))
        scob = jnp.reshape(score[sl, :], (1, 8 * LANES))
        a = (slot_iota == posb).astype(f32) * selb
        src_col = src_col + lax.dot_general(a, idxb, (((1,), (1,)), ((), ())),
                                            preferred_element_type=f32)
        s_col = s_col + lax.dot_general(a, scob, (((1,), (1,)), ((), ())),
                                        preferred_element_type=f32)
    src_row = _col2row(src_col)
    s_row = _col2row(s_col)
    slot_row = lax.broadcasted_iota(i32, (1, KPAD), 1)
    valid_row = slot_row < K
    rank = jnp.zeros((1, KPAD), f32)
    for blk in range(KPAD // BLK):
        sl = slice(blk * BLK, (blk + 1) * BLK)
        v_col = (lax.broadcasted_iota(i32, (BLK, 1), 0) + blk * BLK) < K
        prec = ((s_col[sl, :] > s_row)
                | ((s_col[sl, :] == s_row) & (src_col[sl, :] < src_row))) \
            & v_col
        rank = rank + jnp.sum(prec.astype(f32), axis=0, keepdims=True)
    dest = jnp.where(valid_row, rank.astype(i32), slot_row)
    out = jnp.zeros((KPAD, 1), f32)
    for blk in range(KPAD // BLK):
        sl = slice(blk * BLK, (blk + 1) * BLK)
        a2 = (slot_iota == dest[:, sl]).astype(f32)
        out = out + lax.dot_general(a2, src_row[:, sl],
                                    (((1,), (1,)), ((), ())),
                                    preferred_element_type=f32)
    return out


def _prep_body(resid_ref, logit_ref, res_ref, reg_ref, anc_ref,
               table_ref, srct_ref, srcr_ref):
    resid = resid_ref[0, 0]
    logit = logit_ref[...]
    res = res_ref[...]
    xa, ya, za = anc_ref[0], anc_ref[1], anc_ref[2]
    wa, la, ha, ra = anc_ref[3], anc_ref[4], anc_ref[5], anc_ref[6]
    r0, r1, r2 = reg_ref[0], reg_ref[1], reg_ref[2]
    r3, r4, r5, r6 = reg_ref[3], reg_ref[4], reg_ref[5], reg_ref[6]

    diag = jnp.sqrt(la * la + wa * wa)
    xg = r0 / 10.0 * diag + xa
    yg = r1 / 10.0 * diag + ya
    zg = r2 / 10.0 * ha + za
    wg = jnp.exp(r3 / 5.0) * wa
    lg = jnp.exp(r4 / 5.0) * la
    hg = jnp.exp(r5 / 5.0) * ha
    rg = r6 / 10.0 + ra
    rw = jnp.arctan2(jnp.sin(rg), jnp.cos(rg))
    x2 = r0 / 10.0 * wa + xa
    y2 = r1 / 10.0 * la + ya

    score_t = jax.nn.sigmoid(jax.nn.sigmoid(logit)) + resid
    score_r = res + resid

    row_i = lax.broadcasted_iota(i32, (ROWS, LANES), 0)
    lane_i = lax.broadcasted_iota(i32, (ROWS, LANES), 1)
    flat = row_i * LANES + lane_i
    valid = flat < N
    flat_f = flat.astype(f32)

    for c, v in ((C_XG, xg), (C_YG, yg), (C_ZG, zg), (C_WG, wg), (C_LG, lg),
                 (C_HG, hg), (C_RG, rg), (C_RW, rw), (C_X2, x2), (C_Y2, y2),
                 (C_RA, ra), (C_ST, score_t), (C_SR, score_r), (C_IDX, flat_f),
                 (14, jnp.zeros((ROWS, LANES), f32)),
                 (15, jnp.zeros((ROWS, LANES), f32))):
        table_ref[c] = v

    li_r = lax.broadcasted_iota(i32, (LANES, LANES), 0)
    li_c = lax.broadcasted_iota(i32, (LANES, LANES), 1)
    l_incl = (li_r <= li_c).astype(f32)
    rr = lax.broadcasted_iota(i32, (ROWS, ROWS), 0)
    rc = lax.broadcasted_iota(i32, (ROWS, ROWS), 1)
    l_strict = (rc < rr).astype(f32)
    slot_iota = lax.broadcasted_iota(i32, (KPAD, 1), 0)

    key_t = _sortkey(score_t, valid)
    key_r = _sortkey(score_r, valid)
    tt, tr = _kth_key2(key_t, key_r, K)
    srct_ref[...] = _select_indices(key_t, tt, score_t, flat_f, l_incl,
                                    l_strict, slot_iota).astype(i32)
    srcr_ref[...] = _select_indices(key_r, tr, score_r, flat_f, l_incl,
                                    l_strict, slot_iota).astype(i32)


def _prep(resid, logit, res, reg, anc):
    return pl.pallas_call(
        _prep_body,
        out_shape=[
            jax.ShapeDtypeStruct((NCH, ROWS, LANES), f32),
            jax.ShapeDtypeStruct((KPAD, 1), i32),
            jax.ShapeDtypeStruct((KPAD, 1), i32),
        ],
    )(resid, logit, res, reg, anc)


def _gather_rows(table2d, idx):
    b_total = idx.shape[0]
    nw = 32
    b_per_w = b_total // nw
    mesh = plsc.VectorSubcoreMesh(core_axis_name="c", subcore_axis_name="s")

    @functools.partial(
        pl.kernel, mesh=mesh,
        out_type=jax.ShapeDtypeStruct((b_total, NCH), f32),
        compiler_params=pltpu.CompilerParams(use_tc_tiling_on_sc=False),
        scratch_types=[
            pltpu.VMEM((b_per_w,), i32),
            pltpu.VMEM((b_per_w, NCH), f32),
            pltpu.SemaphoreType.DMA,
        ],
    )
    def k(table_hbm, idx_hbm, out_hbm, idx_v, rows_v, sem):
        wid = lax.axis_index("s") * 2 + lax.axis_index("c")
        base = wid * b_per_w
        pltpu.sync_copy(idx_hbm.at[pl.ds(base, b_per_w)], idx_v)
        pltpu.async_copy(table_hbm.at[idx_v], rows_v, sem).wait()
        pltpu.sync_copy(rows_v, out_hbm.at[pl.ds(base, b_per_w)])

    return k(table2d, idx)


def _row(t, c):
    return t[c:c + 1, :]


def _iou_prec_block(colT, rowpre, blk):
    x1j, x2j, y1j, y2j, areaj, gj = rowpre
    sl = slice(blk * BLK, (blk + 1) * BLK)
    xi = colT[sl, 0:1]
    yi = colT[sl, 1:2]
    wi = colT[sl, 2:3]
    li = colT[sl, 3:4]
    gi = lax.broadcasted_iota(i32, (BLK, 1), 0) + blk * BLK
    x1i = xi - wi * 0.5
    x2i = xi + wi * 0.5
    y1i = yi - li * 0.5
    y2i = yi + li * 0.5
    areai = (x2i - x1i) * (y2i - y1i)
    ix1 = jnp.maximum(x1i, x1j)
    iy1 = jnp.maximum(y1i, y1j)
    ix2 = jnp.minimum(x2i, x2j)
    iy2 = jnp.minimum(y2i, y2j)
    inter = jnp.clip(ix2 - ix1, 0.0) * jnp.clip(iy2 - iy1, 0.0)
    union = areai + areaj - inter
    iou = inter / jnp.maximum(union, 1e-8)
    return ((iou > 0.01) & (gi < gj) & (gi < K)).astype(f32)


def _nms_fixpoint(M_ref, keep_ref):
    keep_ref[...] = jnp.ones((1, KPAD), f32)
    for blk in range(KPAD // BLK):
        sl = slice(blk * BLK, (blk + 1) * BLK)
        alive = keep_ref[0:1, sl]
        md = M_ref[sl, sl]

        def cond(c):
            return c > 0

        def body(c):
            kb = keep_ref[0:1, sl]
            supp = jnp.dot(kb, md, preferred_element_type=f32)
            knew = alive * (supp < 0.5).astype(f32)
            changed = jnp.sum(jnp.abs(knew - kb))
            keep_ref[0:1, sl] = knew
            return (changed > 0.0).astype(i32)

        lax.while_loop(cond, body, i32(1))
        kb = keep_ref[0:1, sl]
        supp_rest = jnp.dot(kb, M_ref[sl, :], preferred_element_type=f32)
        keep_ref[...] = keep_ref[...] * (supp_rest < 0.5).astype(f32)
    return keep_ref[...]


def _rank_of(colsub, s_row, k_row, valid_row):
    g_row = lax.broadcasted_iota(i32, (1, KPAD), 1)
    m_row = jnp.where((k_row > 0.5) & valid_row, s_row, NEG)
    rank = jnp.zeros((1, KPAD), f32)
    for blk in range(KPAD // BLK):
        sl = slice(blk * BLK, (blk + 1) * BLK)
        eye = (lax.broadcasted_iota(i32, (BLK, BLK), 0)
               == lax.broadcasted_iota(i32, (BLK, BLK), 1))
        k_col = jnp.sum(jnp.where(eye, k_row[:, sl], 0.0), axis=1,
                        keepdims=True)
        g_col = lax.broadcasted_iota(i32, (BLK, 1), 0) + blk * BLK
        v_col = g_col < K
        s_col = colsub[sl, 0:1]
        m_col = jnp.where((k_col > 0.5) & v_col, s_col, NEG)
        gt = (m_col > m_row) | ((m_col == m_row) & (g_col < g_row))
        rank = rank + jnp.sum(gt.astype(f32), axis=0, keepdims=True)
    return rank, m_row


def _emit(out_ref, p, rowT, chans, rank, mask_row):
    sel = (lax.broadcasted_iota(i32, (POST + 28, KPAD), 0).astype(f32)
           == rank).astype(f32)
    cols = []
    for c in chans:
        d = _row(rowT, c) * mask_row
        cols.append(jnp.sum(sel * d, axis=1, keepdims=True))
    out_ref[p] = jnp.concatenate(cols, axis=1)


def _nms_body(tc_ref, tr_ref, rc_ref, rr_ref, out_ref, M_ref, keep_ref):
    valid_row = lax.broadcasted_iota(i32, (1, KPAD), 1) < K

    def run(colT_full, rowT, xch, ych, sch, outs):
        colsubM = jnp.concatenate(
            [colT_full[:, xch:xch + 1], colT_full[:, ych:ych + 1],
             colT_full[:, C_WG:C_WG + 1], colT_full[:, C_LG:C_LG + 1]],
            axis=1)
        xj = _row(rowT, xch)
        yj = _row(rowT, ych)
        wj = _row(rowT, C_WG)
        lj = _row(rowT, C_LG)
        sj = _row(rowT, sch)
        gj = lax.broadcasted_iota(i32, (1, KPAD), 1)
        x1j = xj - wj * 0.5
        x2j = xj + wj * 0.5
        y1j = yj - lj * 0.5
        y2j = yj + lj * 0.5
        areaj = (x2j - x1j) * (y2j - y1j)
        rowpre = (x1j, x2j, y1j, y2j, areaj, gj)
        for blk in range(KPAD // BLK):
            sl = slice(blk * BLK, (blk + 1) * BLK)
            M_ref[sl, :] = _iou_prec_block(colsubM, rowpre, blk)
        k_row = _nms_fixpoint(M_ref, keep_ref)
        rank, _ = _rank_of(colT_full[:, sch:sch + 1], sj, k_row, valid_row)
        zj = _row(rowT, C_ZG)
        in_rng = ((xj >= 0.0) & (xj <= 70.4) & (yj >= -40.0) & (yj <= 40.0)
                  & (zj >= -2.2) & (zj <= 0.8))
        mask_row = (in_rng & (k_row > 0.5) & valid_row).astype(f32)
        for p, ang in outs:
            _emit(out_ref, p, rowT,
                  (xch, ych, C_ZG, C_WG, C_LG, C_HG, ang, sch), rank, mask_row)

    tc = tc_ref[...]
    rc = rc_ref[...]
    run(tc, tr_ref[...], C_XG, C_YG, C_ST, ((0, C_RG), (2, C_RW)))
    run(tc, tr_ref[...], C_X2, C_Y2, C_ST, ((1, C_RA),))
    run(rc, rr_ref[...], C_XG, C_YG, C_SR, ((3, C_RW),))


def _nms(tc, tr, rc, rr):
    return pl.pallas_call(
        _nms_body,
        out_shape=jax.ShapeDtypeStruct((4, POST + 28, 8), f32),
        scratch_shapes=[
            pltpu.VMEM((KPAD, KPAD), f32),
            pltpu.VMEM((1, KPAD), f32),
        ],
    )(tc, tr, rc, rr)


def kernel(class_logits, box_regression, anchors, res_scores,
           post_max_into_pre_max):
    resid = (jnp.asarray(post_max_into_pre_max, f32) - 2000.0).reshape(1, 1)
    pad = NPAD - N
    logit = jnp.pad(class_logits[:, 0], (0, pad)).reshape(ROWS, LANES)
    res = jnp.pad(res_scores, (0, pad)).reshape(ROWS, LANES)
    reg = jnp.pad(box_regression, ((0, pad), (0, 0))).T.reshape(7, ROWS, LANES)
    anc = jnp.pad(anchors, ((0, pad), (0, 0))).T.reshape(7, ROWS, LANES)

    table, srct, srcr = _prep(resid, logit, res, reg, anc)
    table2d = table.reshape(NCH, NPAD).T
    idx = jnp.concatenate([srct.reshape(-1), srcr.reshape(-1)])
    rows = _gather_rows(table2d, idx)

    tcol = rows[:KPAD]
    rcol = rows[KPAD:]
    out = _nms(tcol, tcol.T, rcol, rcol.T)
    return out[:, :POST, :]
